# baseline (speedup 1.0000x reference)
import functools

import jax
import jax.numpy as jnp
from jax import lax
from jax.experimental import pallas as pl
from jax.experimental.pallas import tpu as pltpu
from jax.experimental.pallas import tpu_sc as plsc

VOCAB = 100000
V = VOCAB - 1
BATCH = 1024
F = 256
D = 64
W = 2048
NB = (V + W - 1) // W
V_PAD = NB * W
SIM_THRESH = 0.95
L1_THRESH = V / F * SIM_THRESH
NEG_INF = float("-inf")
INT_MAX = 2147483647


def _l2n_cols(x):
    return x * lax.rsqrt(jnp.maximum(jnp.sum(x * x, axis=0, keepdims=True), 1e-12))


BB = 64
OUT_EVERY = 3


def _main_body(selt_ref, rclrt_ref, embt_ref,
               counts_ref, topk_ref, simfs_ref, total_ref, out_ref,
               bv_ref, bi_ref, l1_ref, fs_ref, tot_ref):
    j = pl.program_id(0)

    @pl.when(j == 0)
    def _init():
        bv_ref[...] = jnp.full((F, 1), NEG_INF, jnp.float32)
        bi_ref[...] = jnp.zeros((F, 1), jnp.int32)
        l1_ref[...] = jnp.zeros((F, 1), jnp.float32)
        fs_ref[...] = jnp.zeros((1, 1), jnp.float32)
        tot_ref[...] = jnp.zeros((1, 1), jnp.float32)

    col = j * W + lax.broadcasted_iota(jnp.int32, (1, W), 1)
    valid = col < V

    mask = (rclrt_ref[...] != 0.0).astype(jnp.float32)
    ones_row = jnp.ones((1, BATCH), jnp.float32)
    counts_j = lax.dot_general(ones_row, mask, (((1,), (1,)), ((), ())),
                               preferred_element_type=jnp.float32)
    counts_j = jnp.where(valid, counts_j, 0.0)
    counts_ref[...] = counts_j
    tot_ref[...] = tot_ref[...] + jnp.sum(counts_j, keepdims=True)

    snt = _l2n_cols(selt_ref[...])
    sn = jnp.transpose(snt)
    en = _l2n_cols(embt_ref[...])

    @pl.when((j % OUT_EVERY == 0) & (j < OUT_EVERY * (BATCH // BB)))
    def _bcast():
        out_ref[...] = jnp.broadcast_to(snt[None], (BB, D, F))
    fm = lax.dot_general(sn, en, (((1,), (0,)), ((), ())))
    fmv = jnp.where(valid, fm, 0.0)

    l1_ref[...] = l1_ref[...] + jnp.sum(
        jnp.maximum(fmv, 0.0), axis=1, keepdims=True)

    l2 = jnp.sum(jnp.where(fmv > SIM_THRESH, fmv, 0.0),
                 axis=0, keepdims=True)
    fs_ref[...] = fs_ref[...] + jnp.sum(
        jnp.where(l2 > 1.0, l2, 0.0), keepdims=True)

    cm = (counts_j != 0.0).astype(jnp.float32)
    filtered = jnp.where(valid, fm * cm, NEG_INF)
    lmax = jnp.max(filtered, axis=1, keepdims=True)
    lidx = jnp.min(jnp.where(filtered == lmax, col, INT_MAX),
                   axis=1, keepdims=True)
    bv = bv_ref[...]
    upd = lmax > bv
    bv_ref[...] = jnp.where(upd, lmax, bv)
    bi_ref[...] = jnp.where(upd, lidx, bi_ref[...])

    @pl.when(j == NB - 1)
    def _fin():
        l1 = l1_ref[...]
        sim = jnp.sum(l1 * (l1 < L1_THRESH).astype(jnp.float32), keepdims=True)
        simfs = sim + fs_ref[...]
        simfs_ref[...] = jnp.broadcast_to(simfs, (1, 16))
        total_ref[...] = jnp.broadcast_to(tot_ref[...], (1, 16))
        topk_ref[...] = bi_ref[...]


def _sc_loss_body(counts_hbm, topk_hbm, simfs_hbm, total_hbm, loss_hbm,
                  counts_v, idx_v, simfs_v, total_v, out_v):
    @pl.when((lax.axis_index("c") == 0) & (lax.axis_index("s") == 0))
    def _():
        pltpu.sync_copy(counts_hbm, counts_v)
        pltpu.sync_copy(topk_hbm, idx_v)
        pltpu.sync_copy(simfs_hbm, simfs_v)
        pltpu.sync_copy(total_hbm, total_v)

        def body(i, acc):
            idx = idx_v[pl.ds(i * 16, 16)]
            return acc + plsc.load_gather(counts_v, [idx])

        acc = lax.fori_loop(0, F // 16, body, jnp.zeros((16,), jnp.float32))
        filt = jnp.sum(acc)
        tv = total_v[...]
        out_v[...] = simfs_v[...] + 0.1 * ((tv - filt) / tv)
        pltpu.sync_copy(out_v, loss_hbm)


def _run_sc_loss(counts1, topk1, simfs16, total16):
    sc_loss = functools.partial(
        pl.kernel,
        out_type=jax.ShapeDtypeStruct((16,), jnp.float32),
        mesh=plsc.VectorSubcoreMesh(core_axis_name="c", subcore_axis_name="s"),
        compiler_params=pltpu.CompilerParams(needs_layout_passes=False),
        scratch_types=[
            pltpu.VMEM((V_PAD,), jnp.float32),
            pltpu.VMEM((F,), jnp.int32),
            pltpu.VMEM((16,), jnp.float32),
            pltpu.VMEM((16,), jnp.float32),
            pltpu.VMEM((16,), jnp.float32),
        ],
    )(_sc_loss_body)
    return sc_loss(counts1, topk1, simfs16, total16)


def kernel(rclr, embeddings, selectors):
    rclr_t = rclr.T
    emb_t = embeddings.T
    sel_t = selectors.T

    counts2, topk2, simfs2, total2, out_t = pl.pallas_call(
        _main_body,
        grid=(NB,),
        in_specs=[
            pl.BlockSpec((D, F), lambda j: (0, 0)),
            pl.BlockSpec((W, BATCH), lambda j: (j, 0)),
            pl.BlockSpec((D, W), lambda j: (0, j)),
        ],
        out_specs=[
            pl.BlockSpec((1, W), lambda j: (0, j)),
            pl.BlockSpec((F, 1), lambda j: (0, 0)),
            pl.BlockSpec((1, 16), lambda j: (0, 0)),
            pl.BlockSpec((1, 16), lambda j: (0, 0)),
            pl.BlockSpec((BB, D, F),
                         lambda j: (jnp.minimum(j // OUT_EVERY,
                                                BATCH // BB - 1), 0, 0)),
        ],
        out_shape=[
            jax.ShapeDtypeStruct((1, V_PAD), jnp.float32),
            jax.ShapeDtypeStruct((F, 1), jnp.int32),
            jax.ShapeDtypeStruct((1, 16), jnp.float32),
            jax.ShapeDtypeStruct((1, 16), jnp.float32),
            jax.ShapeDtypeStruct((BATCH, D, F), jnp.float32),
        ],
        scratch_shapes=[
            pltpu.VMEM((F, 1), jnp.float32),
            pltpu.VMEM((F, 1), jnp.int32),
            pltpu.VMEM((F, 1), jnp.float32),
            pltpu.VMEM((1, 1), jnp.float32),
            pltpu.VMEM((1, 1), jnp.float32),
        ],
    )(sel_t, rclr_t, emb_t)

    out_emb = out_t.transpose(0, 2, 1)
    top_k = topk2.reshape(F)
    loss16 = _run_sc_loss(counts2.reshape(V_PAD), top_k,
                          simfs2.reshape(16), total2.reshape(16))
    loss = loss16[0]

    return (out_emb, top_k, loss)


# speedup vs baseline: 1.1066x; 1.1066x over previous
import functools

import jax
import jax.numpy as jnp
from jax import lax
from jax.experimental import pallas as pl
from jax.experimental.pallas import tpu as pltpu
from jax.experimental.pallas import tpu_sc as plsc

VOCAB = 100000
V = VOCAB - 1
BATCH = 1024
F = 256
D = 64
W = 4096
NB = (V + W - 1) // W
V_PAD = NB * W
SIM_THRESH = 0.95
L1_THRESH = V / F * SIM_THRESH
NEG_INF = float("-inf")
INT_MAX = 2147483647


def _l2n_cols(x):
    return x * lax.rsqrt(jnp.maximum(jnp.sum(x * x, axis=0, keepdims=True), 1e-12))


def _main_body(selt_ref, rclrt_ref, embt_ref,
               counts_ref, topk_ref, simfs_ref, total_ref,
               bv_ref, bi_ref, l1_ref, fs_ref, tot_ref):
    j = pl.program_id(0)

    @pl.when(j == 0)
    def _init():
        bv_ref[...] = jnp.full((F, 1), NEG_INF, jnp.float32)
        bi_ref[...] = jnp.zeros((F, 1), jnp.int32)
        l1_ref[...] = jnp.zeros((F, 1), jnp.float32)
        fs_ref[...] = jnp.zeros((1, 1), jnp.float32)
        tot_ref[...] = jnp.zeros((1, 1), jnp.float32)

    col = j * W + lax.broadcasted_iota(jnp.int32, (1, W), 1)
    valid = col < V

    mask = (rclrt_ref[...] != 0.0).astype(jnp.float32)
    ones_row = jnp.ones((1, BATCH), jnp.float32)
    counts_j = lax.dot_general(ones_row, mask, (((1,), (1,)), ((), ())),
                               preferred_element_type=jnp.float32)
    counts_j = jnp.where(valid, counts_j, 0.0)
    counts_ref[...] = counts_j
    tot_ref[...] = tot_ref[...] + jnp.sum(counts_j, keepdims=True)

    sn = jnp.transpose(_l2n_cols(selt_ref[...]))
    en = _l2n_cols(embt_ref[...])
    fm = lax.dot_general(sn, en, (((1,), (0,)), ((), ())))
    fmv = jnp.where(valid, fm, 0.0)

    l1_ref[...] = l1_ref[...] + jnp.sum(
        jnp.maximum(fmv, 0.0), axis=1, keepdims=True)

    l2 = jnp.sum(jnp.where(fmv > SIM_THRESH, fmv, 0.0),
                 axis=0, keepdims=True)
    fs_ref[...] = fs_ref[...] + jnp.sum(
        jnp.where(l2 > 1.0, l2, 0.0), keepdims=True)

    cm = (counts_j != 0.0).astype(jnp.float32)
    filtered = jnp.where(valid, fm * cm, NEG_INF)
    lmax = jnp.max(filtered, axis=1, keepdims=True)
    lidx = jnp.min(jnp.where(filtered == lmax, col, INT_MAX),
                   axis=1, keepdims=True)
    bv = bv_ref[...]
    upd = lmax > bv
    bv_ref[...] = jnp.where(upd, lmax, bv)
    bi_ref[...] = jnp.where(upd, lidx, bi_ref[...])

    @pl.when(j == NB - 1)
    def _fin():
        l1 = l1_ref[...]
        sim = jnp.sum(l1 * (l1 < L1_THRESH).astype(jnp.float32), keepdims=True)
        simfs = sim + fs_ref[...]
        simfs_ref[...] = jnp.broadcast_to(simfs, (1, 16))
        total_ref[...] = jnp.broadcast_to(tot_ref[...], (1, 16))
        topk_ref[...] = bi_ref[...]


def _bcast_body(selt_ref, out_ref):
    snt = _l2n_cols(selt_ref[...])
    out_ref[...] = jnp.broadcast_to(snt[None], out_ref.shape)


def _sc_loss_body(counts_hbm, topk_hbm, simfs_hbm, total_hbm, loss_hbm,
                  counts_v, idx_v, simfs_v, total_v, out_v):
    @pl.when((lax.axis_index("c") == 0) & (lax.axis_index("s") == 0))
    def _():
        pltpu.sync_copy(counts_hbm, counts_v)
        pltpu.sync_copy(topk_hbm, idx_v)
        pltpu.sync_copy(simfs_hbm, simfs_v)
        pltpu.sync_copy(total_hbm, total_v)

        def body(i, acc):
            idx = idx_v[pl.ds(i * 16, 16)]
            return acc + plsc.load_gather(counts_v, [idx])

        acc = lax.fori_loop(0, F // 16, body, jnp.zeros((16,), jnp.float32))
        filt = jnp.sum(acc)
        tv = total_v[...]
        out_v[...] = simfs_v[...] + 0.1 * ((tv - filt) / tv)
        pltpu.sync_copy(out_v, loss_hbm)


def _run_sc_loss(counts1, topk1, simfs16, total16):
    sc_loss = functools.partial(
        pl.kernel,
        out_type=jax.ShapeDtypeStruct((16,), jnp.float32),
        mesh=plsc.VectorSubcoreMesh(core_axis_name="c", subcore_axis_name="s"),
        compiler_params=pltpu.CompilerParams(needs_layout_passes=False),
        scratch_types=[
            pltpu.VMEM((V_PAD,), jnp.float32),
            pltpu.VMEM((F,), jnp.int32),
            pltpu.VMEM((16,), jnp.float32),
            pltpu.VMEM((16,), jnp.float32),
            pltpu.VMEM((16,), jnp.float32),
        ],
    )(_sc_loss_body)
    return sc_loss(counts1, topk1, simfs16, total16)


def kernel(rclr, embeddings, selectors):
    rclr_t = rclr.T
    emb_t = embeddings.T
    sel_t = selectors.T

    counts2, topk2, simfs2, total2 = pl.pallas_call(
        _main_body,
        grid=(NB,),
        in_specs=[
            pl.BlockSpec((D, F), lambda j: (0, 0)),
            pl.BlockSpec((W, BATCH), lambda j: (j, 0)),
            pl.BlockSpec((D, W), lambda j: (0, j)),
        ],
        out_specs=[
            pl.BlockSpec((1, W), lambda j: (0, j)),
            pl.BlockSpec((F, 1), lambda j: (0, 0)),
            pl.BlockSpec((1, 16), lambda j: (0, 0)),
            pl.BlockSpec((1, 16), lambda j: (0, 0)),
        ],
        out_shape=[
            jax.ShapeDtypeStruct((1, V_PAD), jnp.float32),
            jax.ShapeDtypeStruct((F, 1), jnp.int32),
            jax.ShapeDtypeStruct((1, 16), jnp.float32),
            jax.ShapeDtypeStruct((1, 16), jnp.float32),
        ],
        scratch_shapes=[
            pltpu.VMEM((F, 1), jnp.float32),
            pltpu.VMEM((F, 1), jnp.int32),
            pltpu.VMEM((F, 1), jnp.float32),
            pltpu.VMEM((1, 1), jnp.float32),
            pltpu.VMEM((1, 1), jnp.float32),
        ],
    )(sel_t, rclr_t, emb_t)

    top_k = topk2.reshape(F)
    loss16 = _run_sc_loss(counts2.reshape(V_PAD), top_k,
                          simfs2.reshape(16), total2.reshape(16))
    loss = loss16[0]

    BB = 64
    out_t = pl.pallas_call(
        _bcast_body,
        grid=(BATCH // BB,),
        in_specs=[pl.BlockSpec((D, F), lambda i: (0, 0))],
        out_specs=pl.BlockSpec((BB, D, F), lambda i: (i, 0, 0)),
        out_shape=jax.ShapeDtypeStruct((BATCH, D, F), jnp.float32),
    )(sel_t)
    out_emb = out_t.transpose(0, 2, 1)

    return (out_emb, top_k, loss)


# speedup vs baseline: 1.1153x; 1.0079x over previous
import functools

import jax
import jax.numpy as jnp
from jax import lax
from jax.experimental import pallas as pl
from jax.experimental.pallas import tpu as pltpu
from jax.experimental.pallas import tpu_sc as plsc

VOCAB = 100000
V = VOCAB - 1
BATCH = 1024
F = 256
D = 64
W = 5120
NB = (V + W - 1) // W
V_PAD = NB * W
SIM_THRESH = 0.95
L1_THRESH = V / F * SIM_THRESH
NEG_INF = float("-inf")
INT_MAX = 2147483647


def _l2n_cols(x):
    return x * lax.rsqrt(jnp.maximum(jnp.sum(x * x, axis=0, keepdims=True), 1e-12))


def _main_body(selt_ref, rclrt_ref, embt_ref,
               counts_ref, topk_ref, simfs_ref, total_ref,
               bv_ref, bi_ref, l1_ref, fs_ref, tot_ref):
    j = pl.program_id(0)

    @pl.when(j == 0)
    def _init():
        bv_ref[...] = jnp.full((F, 1), NEG_INF, jnp.float32)
        bi_ref[...] = jnp.zeros((F, 1), jnp.int32)
        l1_ref[...] = jnp.zeros((F, 1), jnp.float32)
        fs_ref[...] = jnp.zeros((1, 1), jnp.float32)
        tot_ref[...] = jnp.zeros((1, 1), jnp.float32)

    col = j * W + lax.broadcasted_iota(jnp.int32, (1, W), 1)
    valid = col < V

    mask = (rclrt_ref[...] != 0.0).astype(jnp.float32)
    ones_row = jnp.ones((1, BATCH), jnp.float32)
    counts_j = lax.dot_general(ones_row, mask, (((1,), (1,)), ((), ())),
                               preferred_element_type=jnp.float32)
    counts_j = jnp.where(valid, counts_j, 0.0)
    counts_ref[...] = counts_j
    tot_ref[...] = tot_ref[...] + jnp.sum(counts_j, keepdims=True)

    sn = jnp.transpose(_l2n_cols(selt_ref[...]))
    en = _l2n_cols(embt_ref[...])
    fm = lax.dot_general(sn, en, (((1,), (0,)), ((), ())))
    fmv = jnp.where(valid, fm, 0.0)

    l1_ref[...] = l1_ref[...] + jnp.sum(
        jnp.maximum(fmv, 0.0), axis=1, keepdims=True)

    l2 = jnp.sum(jnp.where(fmv > SIM_THRESH, fmv, 0.0),
                 axis=0, keepdims=True)
    fs_ref[...] = fs_ref[...] + jnp.sum(
        jnp.where(l2 > 1.0, l2, 0.0), keepdims=True)

    cm = (counts_j != 0.0).astype(jnp.float32)
    filtered = jnp.where(valid, fm * cm, NEG_INF)
    lmax = jnp.max(filtered, axis=1, keepdims=True)
    lidx = jnp.min(jnp.where(filtered == lmax, col, INT_MAX),
                   axis=1, keepdims=True)
    bv = bv_ref[...]
    upd = lmax > bv
    bv_ref[...] = jnp.where(upd, lmax, bv)
    bi_ref[...] = jnp.where(upd, lidx, bi_ref[...])

    @pl.when(j == NB - 1)
    def _fin():
        l1 = l1_ref[...]
        sim = jnp.sum(l1 * (l1 < L1_THRESH).astype(jnp.float32), keepdims=True)
        simfs = sim + fs_ref[...]
        simfs_ref[...] = jnp.broadcast_to(simfs, (1, 16))
        total_ref[...] = jnp.broadcast_to(tot_ref[...], (1, 16))
        topk_ref[...] = bi_ref[...]


def _bcast_body(selt_ref, out_ref):
    snt = _l2n_cols(selt_ref[...])
    out_ref[...] = jnp.broadcast_to(snt[None], out_ref.shape)


def _sc_loss_body(counts_hbm, topk_hbm, simfs_hbm, total_hbm, loss_hbm,
                  counts_v, idx_v, simfs_v, total_v, out_v):
    @pl.when((lax.axis_index("c") == 0) & (lax.axis_index("s") == 0))
    def _():
        pltpu.sync_copy(counts_hbm, counts_v)
        pltpu.sync_copy(topk_hbm, idx_v)
        pltpu.sync_copy(simfs_hbm, simfs_v)
        pltpu.sync_copy(total_hbm, total_v)

        def body(i, acc):
            idx = idx_v[pl.ds(i * 16, 16)]
            return acc + plsc.load_gather(counts_v, [idx])

        acc = lax.fori_loop(0, F // 16, body, jnp.zeros((16,), jnp.float32))
        filt = jnp.sum(acc)
        tv = total_v[...]
        out_v[...] = simfs_v[...] + 0.1 * ((tv - filt) / tv)
        pltpu.sync_copy(out_v, loss_hbm)


def _run_sc_loss(counts1, topk1, simfs16, total16):
    sc_loss = functools.partial(
        pl.kernel,
        out_type=jax.ShapeDtypeStruct((16,), jnp.float32),
        mesh=plsc.VectorSubcoreMesh(core_axis_name="c", subcore_axis_name="s"),
        compiler_params=pltpu.CompilerParams(needs_layout_passes=False),
        scratch_types=[
            pltpu.VMEM((V_PAD,), jnp.float32),
            pltpu.VMEM((F,), jnp.int32),
            pltpu.VMEM((16,), jnp.float32),
            pltpu.VMEM((16,), jnp.float32),
            pltpu.VMEM((16,), jnp.float32),
        ],
    )(_sc_loss_body)
    return sc_loss(counts1, topk1, simfs16, total16)


def kernel(rclr, embeddings, selectors):
    rclr_t = rclr.T
    emb_t = embeddings.T
    sel_t = selectors.T

    counts2, topk2, simfs2, total2 = pl.pallas_call(
        _main_body,
        grid=(NB,),
        in_specs=[
            pl.BlockSpec((D, F), lambda j: (0, 0)),
            pl.BlockSpec((W, BATCH), lambda j: (j, 0)),
            pl.BlockSpec((D, W), lambda j: (0, j)),
        ],
        out_specs=[
            pl.BlockSpec((1, W), lambda j: (0, j)),
            pl.BlockSpec((F, 1), lambda j: (0, 0)),
            pl.BlockSpec((1, 16), lambda j: (0, 0)),
            pl.BlockSpec((1, 16), lambda j: (0, 0)),
        ],
        out_shape=[
            jax.ShapeDtypeStruct((1, V_PAD), jnp.float32),
            jax.ShapeDtypeStruct((F, 1), jnp.int32),
            jax.ShapeDtypeStruct((1, 16), jnp.float32),
            jax.ShapeDtypeStruct((1, 16), jnp.float32),
        ],
        scratch_shapes=[
            pltpu.VMEM((F, 1), jnp.float32),
            pltpu.VMEM((F, 1), jnp.int32),
            pltpu.VMEM((F, 1), jnp.float32),
            pltpu.VMEM((1, 1), jnp.float32),
            pltpu.VMEM((1, 1), jnp.float32),
        ],
    )(sel_t, rclr_t, emb_t)

    top_k = topk2.reshape(F)
    loss16 = _run_sc_loss(counts2.reshape(V_PAD), top_k,
                          simfs2.reshape(16), total2.reshape(16))
    loss = loss16[0]

    BB = 64
    out_t = pl.pallas_call(
        _bcast_body,
        grid=(BATCH // BB,),
        in_specs=[pl.BlockSpec((D, F), lambda i: (0, 0))],
        out_specs=pl.BlockSpec((BB, D, F), lambda i: (i, 0, 0)),
        out_shape=jax.ShapeDtypeStruct((BATCH, D, F), jnp.float32),
    )(sel_t)
    out_emb = out_t.transpose(0, 2, 1)

    return (out_emb, top_k, loss)
